# baseline (speedup 1.0000x reference)
import jax
import jax.numpy as jnp
from jax.experimental import pallas as pl
from jax.experimental.pallas import tpu as pltpu

_CT_CHUNKS = 5
_SV_CHUNKS = 8


def _body(s_ref, t_ref, ct_in, ind_in, new_in, sv_in,
          ct_out, ind_out, sv_out,
          ism, csm, ct_sem, sv_sem, ind_sem, sm_sem):
    idx = s_ref[0]
    sidx = s_ref[1]
    ibase = pl.multiple_of(s_ref[2], 128)
    ioff = idx - ibase
    n_rows = ct_in.shape[0] // _CT_CHUNKS
    ct_copies = [
        pltpu.make_async_copy(
            ct_in.at[pl.ds(i * n_rows, n_rows)],
            ct_out.at[pl.ds(i * n_rows, n_rows)],
            ct_sem,
        )
        for i in range(_CT_CHUNKS)
    ]
    n_slots = sv_in.shape[0] // _SV_CHUNKS
    sv_copies = [
        pltpu.make_async_copy(
            sv_in.at[pl.ds(i * n_slots, n_slots)],
            sv_out.at[pl.ds(i * n_slots, n_slots)],
            sv_sem,
        )
        for i in range(_SV_CHUNKS)
    ]
    for c in ct_copies:
        c.start()
    for c in sv_copies:
        c.start()
    ind_copy = pltpu.make_async_copy(ind_in, ind_out, ind_sem)
    ind_copy.start()
    cur_rd = pltpu.make_async_copy(ind_in.at[pl.ds(ibase, 128)], ism, sm_sem)
    cur_rd.start()
    cur_rd.wait()
    cur = ism[ioff]
    ism[ioff] = cur + 1
    curb = pl.multiple_of((cur // 8) * 8, 8)
    ct_rd = pltpu.make_async_copy(
        ct_in.at[pl.ds(curb, 8), pl.ds(ibase, 128)], csm, sm_sem)
    ct_rd.start()
    ind_copy.wait()
    ind_patch = pltpu.make_async_copy(ism, ind_out.at[pl.ds(ibase, 128)], sm_sem)
    ind_patch.start()
    ind_patch.wait()
    ct_rd.wait()
    blk = csm[...]
    ri = jax.lax.broadcasted_iota(jnp.int32, blk.shape, 0)
    ci = jax.lax.broadcasted_iota(jnp.int32, blk.shape, 1)
    csm[...] = jnp.where((ri == cur - curb) & (ci == ioff), t_ref[0], blk)
    for c in ct_copies:
        c.wait()
    ct_patch = pltpu.make_async_copy(
        csm, ct_out.at[pl.ds(curb, 8), pl.ds(ibase, 128)], sm_sem)
    ct_patch.start()
    ct_patch.wait()
    for c in sv_copies:
        c.wait()
    sv_patch = pltpu.make_async_copy(new_in.at[sidx], sv_out.at[sidx], sv_sem)
    sv_patch.start()
    sv_patch.wait()


def kernel(clicktimes, indices, idx, t, saved, new, save_index):
    n_det, max_clicks = clicktimes.shape
    idx = jnp.asarray(idx, jnp.int32)
    sidx = jnp.asarray(save_index, jnp.int32)
    ct_t = clicktimes.T
    sv_t = jnp.transpose(saved, (0, 2, 1))
    new_t = jnp.transpose(new, (0, 2, 1))
    s = jnp.stack([idx, sidx, (idx // 128) * 128])
    t_arr = jnp.reshape(t, (1,))

    grid_spec = pltpu.PrefetchScalarGridSpec(
        num_scalar_prefetch=1,
        grid=(1,),
        in_specs=[
            pl.BlockSpec(memory_space=pltpu.SMEM),
            pl.BlockSpec(memory_space=pl.ANY),
            pl.BlockSpec(memory_space=pl.ANY),
            pl.BlockSpec(memory_space=pl.ANY),
            pl.BlockSpec(memory_space=pl.ANY),
        ],
        out_specs=[
            pl.BlockSpec(memory_space=pl.ANY),
            pl.BlockSpec(memory_space=pl.ANY),
            pl.BlockSpec(memory_space=pl.ANY),
        ],
        scratch_shapes=[
            pltpu.SMEM((128,), jnp.int32),
            pltpu.VMEM((8, 128), jnp.float32),
            pltpu.SemaphoreType.DMA,
            pltpu.SemaphoreType.DMA,
            pltpu.SemaphoreType.DMA,
            pltpu.SemaphoreType.DMA,
        ],
    )
    ct_out, ind_out, sv_out = pl.pallas_call(
        _body,
        grid_spec=grid_spec,
        out_shape=[
            jax.ShapeDtypeStruct(ct_t.shape, ct_t.dtype),
            jax.ShapeDtypeStruct(indices.shape, indices.dtype),
            jax.ShapeDtypeStruct(sv_t.shape, sv_t.dtype),
        ],
        compiler_params=pltpu.CompilerParams(
            dimension_semantics=("arbitrary",),
        ),
    )(s, t_arr, ct_t, indices, new_t, sv_t)
    return (
        ct_out.T,
        ind_out,
        jnp.transpose(sv_out, (0, 2, 1)),
        save_index + 1,
    )


# speedup vs baseline: 33.4366x; 33.4366x over previous
import jax
import jax.numpy as jnp
from jax import lax
from jax.experimental import pallas as pl
from jax.experimental.pallas import tpu as pltpu
from jax.experimental.pallas import tpu_sc as plsc

_LANES = 128
_CT_BW = 2048
_SV_BS = 4


def _sc_ind_body(par_ref, ind_ref, out_ref, pv, iv):
    cid = lax.axis_index("c")
    sid = lax.axis_index("s")

    @pl.when(jnp.logical_and(cid == 0, sid == 0))
    def _():
        pltpu.sync_copy(par_ref, pv)
        idxv = pv[...]
        pltpu.sync_copy(ind_ref, iv)
        lane = lax.iota(jnp.int32, 16)
        plsc.addupdate_scatter(
            iv, [idxv], jnp.full((16,), 1, jnp.int32), mask=lane == 0
        )
        pltpu.sync_copy(iv, out_ref)


def _make_sc_ind(n_det):
    mesh = plsc.VectorSubcoreMesh(
        core_axis_name="c", subcore_axis_name="s", num_cores=2, num_subcores=16
    )
    return pl.kernel(
        _sc_ind_body,
        out_type=jax.ShapeDtypeStruct((n_det,), jnp.int32),
        mesh=mesh,
        scratch_types=[
            pltpu.VMEM((16,), jnp.int32),
            pltpu.VMEM((n_det,), jnp.int32),
        ],
        compiler_params=pltpu.CompilerParams(needs_layout_passes=False),
    )


def _ct_body(s_ref, ct_in, ind_in, t_ref, ct_out, ism, sm_sem):
    i = pl.program_id(0)

    @pl.when(i == 0)
    def _():
        ibase = pl.multiple_of(s_ref[1], _LANES)
        cur_rd = pltpu.make_async_copy(
            ind_in.at[pl.ds(ibase, _LANES)], ism, sm_sem)
        cur_rd.start()
        cur_rd.wait()

    idx = s_ref[0]
    cur = ism[idx - pl.multiple_of(s_ref[1], _LANES)]
    blk = ct_in[...]
    ri = jax.lax.broadcasted_iota(jnp.int32, blk.shape, 0)
    ci = jax.lax.broadcasted_iota(jnp.int32, blk.shape, 1) + i * _CT_BW
    ct_out[...] = jnp.where((ri == cur) & (ci == idx), t_ref[0], blk)


def _sv_body(s_ref, sv_in, new_in, sv_out):
    i = pl.program_id(0)
    sidx = s_ref[0]
    slot = jax.lax.broadcasted_iota(jnp.int32, sv_in.shape, 0) + i * _SV_BS
    sv_out[...] = jnp.where(slot == sidx, new_in[...], sv_in[...])


def kernel(clicktimes, indices, idx, t, saved, new, save_index):
    n_det, max_clicks = clicktimes.shape
    n_save, batch, dim = saved.shape
    idx = jnp.asarray(idx, jnp.int32)
    sidx = jnp.asarray(save_index, jnp.int32)
    ct_t = clicktimes.T
    sv_t = jnp.transpose(saved, (0, 2, 1))
    new_t = jnp.transpose(new, (0, 2, 1))
    t_arr = jnp.reshape(t, (1,))
    par16 = jnp.full((16,), idx, jnp.int32)

    ind_out = _make_sc_ind(n_det)(par16, indices)

    ct_grid = pl.cdiv(n_det, _CT_BW)
    ct_spec = pltpu.PrefetchScalarGridSpec(
        num_scalar_prefetch=1,
        grid=(ct_grid,),
        in_specs=[
            pl.BlockSpec((max_clicks, _CT_BW), lambda i, s: (0, i)),
            pl.BlockSpec(memory_space=pl.ANY),
            pl.BlockSpec(memory_space=pltpu.SMEM),
        ],
        out_specs=pl.BlockSpec((max_clicks, _CT_BW), lambda i, s: (0, i)),
        scratch_shapes=[
            pltpu.SMEM((_LANES,), jnp.int32),
            pltpu.SemaphoreType.DMA,
        ],
    )
    ct_out = pl.pallas_call(
        _ct_body,
        grid_spec=ct_spec,
        out_shape=jax.ShapeDtypeStruct(ct_t.shape, ct_t.dtype),
        compiler_params=pltpu.CompilerParams(
            dimension_semantics=("arbitrary",),
        ),
    )(jnp.stack([idx, (idx // _LANES) * _LANES]), ct_t, indices, t_arr)

    sv_spec = pltpu.PrefetchScalarGridSpec(
        num_scalar_prefetch=1,
        grid=(n_save // _SV_BS,),
        in_specs=[
            pl.BlockSpec((_SV_BS, dim, batch), lambda i, s: (i, 0, 0)),
            pl.BlockSpec((1, dim, batch), lambda i, s: (s[1], 0, 0)),
        ],
        out_specs=pl.BlockSpec((_SV_BS, dim, batch), lambda i, s: (i, 0, 0)),
    )
    sv_out = pl.pallas_call(
        _sv_body,
        grid_spec=sv_spec,
        out_shape=jax.ShapeDtypeStruct(sv_t.shape, sv_t.dtype),
        compiler_params=pltpu.CompilerParams(
            dimension_semantics=("arbitrary",),
        ),
    )(jnp.stack([sidx, sidx]), sv_t, new_t)

    return (
        ct_out.T,
        ind_out,
        jnp.transpose(sv_out, (0, 2, 1)),
        save_index + 1,
    )


# speedup vs baseline: 38.6613x; 1.1563x over previous
import jax
import jax.numpy as jnp
from jax import lax
from jax.experimental import pallas as pl
from jax.experimental.pallas import tpu as pltpu
from jax.experimental.pallas import tpu_sc as plsc

_LANES = 128
_CT_BW = 8192
_SV_BS = 8


def _sc_ind_body(par_ref, ind_ref, out_ref, pv, iv):
    cid = lax.axis_index("c")
    sid = lax.axis_index("s")

    @pl.when(jnp.logical_and(cid == 0, sid == 0))
    def _():
        pltpu.sync_copy(par_ref, pv)
        idxv = pv[...]
        pltpu.sync_copy(ind_ref, iv)
        lane = lax.iota(jnp.int32, 16)
        plsc.addupdate_scatter(
            iv, [idxv], jnp.full((16,), 1, jnp.int32), mask=lane == 0
        )
        pltpu.sync_copy(iv, out_ref)


def _make_sc_ind(n_det):
    mesh = plsc.VectorSubcoreMesh(
        core_axis_name="c", subcore_axis_name="s", num_cores=2, num_subcores=16
    )
    return pl.kernel(
        _sc_ind_body,
        out_type=jax.ShapeDtypeStruct((n_det,), jnp.int32),
        mesh=mesh,
        scratch_types=[
            pltpu.VMEM((16,), jnp.int32),
            pltpu.VMEM((n_det,), jnp.int32),
        ],
        compiler_params=pltpu.CompilerParams(needs_layout_passes=False),
    )


def _ct_body(s_ref, ct_in, ind_in, t_ref, ct_out, ism, sm_sem):
    i = pl.program_id(0)

    @pl.when(i == 0)
    def _():
        ibase = pl.multiple_of(s_ref[1], _LANES)
        cur_rd = pltpu.make_async_copy(
            ind_in.at[pl.ds(ibase, _LANES)], ism, sm_sem)
        cur_rd.start()
        cur_rd.wait()

    idx = s_ref[0]

    @pl.when(i != idx // _CT_BW)
    def _():
        ct_out[...] = ct_in[...]

    @pl.when(i == idx // _CT_BW)
    def _():
        cur = ism[idx - pl.multiple_of(s_ref[1], _LANES)]
        blk = ct_in[...]
        ri = jax.lax.broadcasted_iota(jnp.int32, blk.shape, 0)
        ci = jax.lax.broadcasted_iota(jnp.int32, blk.shape, 1) + i * _CT_BW
        ct_out[...] = jnp.where((ri == cur) & (ci == idx), t_ref[0], blk)


def _sv_body(s_ref, sv_in, new_in, sv_out):
    i = pl.program_id(0)
    sidx = s_ref[0]

    @pl.when(i != sidx // _SV_BS)
    def _():
        sv_out[...] = sv_in[...]

    @pl.when(i == sidx // _SV_BS)
    def _():
        slot = jax.lax.broadcasted_iota(jnp.int32, sv_in.shape, 0) + i * _SV_BS
        sv_out[...] = jnp.where(slot == sidx, new_in[...], sv_in[...])


def kernel(clicktimes, indices, idx, t, saved, new, save_index):
    n_det, max_clicks = clicktimes.shape
    n_save, batch, dim = saved.shape
    idx = jnp.asarray(idx, jnp.int32)
    sidx = jnp.asarray(save_index, jnp.int32)
    ct_t = clicktimes.T
    sv_t = jnp.transpose(saved, (0, 2, 1))
    new_t = jnp.transpose(new, (0, 2, 1))
    t_arr = jnp.reshape(t, (1,))
    par16 = jnp.full((16,), idx, jnp.int32)

    ind_out = _make_sc_ind(n_det)(par16, indices)

    ct_grid = pl.cdiv(n_det, _CT_BW)
    ct_spec = pltpu.PrefetchScalarGridSpec(
        num_scalar_prefetch=1,
        grid=(ct_grid,),
        in_specs=[
            pl.BlockSpec((max_clicks, _CT_BW), lambda i, s: (0, i)),
            pl.BlockSpec(memory_space=pl.ANY),
            pl.BlockSpec(memory_space=pltpu.SMEM),
        ],
        out_specs=pl.BlockSpec((max_clicks, _CT_BW), lambda i, s: (0, i)),
        scratch_shapes=[
            pltpu.SMEM((_LANES,), jnp.int32),
            pltpu.SemaphoreType.DMA,
        ],
    )
    ct_out = pl.pallas_call(
        _ct_body,
        grid_spec=ct_spec,
        out_shape=jax.ShapeDtypeStruct(ct_t.shape, ct_t.dtype),
        compiler_params=pltpu.CompilerParams(
            dimension_semantics=("arbitrary",),
            vmem_limit_bytes=128 * 1024 * 1024,
        ),
    )(jnp.stack([idx, (idx // _LANES) * _LANES]), ct_t, indices, t_arr)

    sv_spec = pltpu.PrefetchScalarGridSpec(
        num_scalar_prefetch=1,
        grid=(n_save // _SV_BS,),
        in_specs=[
            pl.BlockSpec((_SV_BS, dim, batch), lambda i, s: (i, 0, 0)),
            pl.BlockSpec((1, dim, batch), lambda i, s: (s[1], 0, 0)),
        ],
        out_specs=pl.BlockSpec((_SV_BS, dim, batch), lambda i, s: (i, 0, 0)),
    )
    sv_out = pl.pallas_call(
        _sv_body,
        grid_spec=sv_spec,
        out_shape=jax.ShapeDtypeStruct(sv_t.shape, sv_t.dtype),
        compiler_params=pltpu.CompilerParams(
            dimension_semantics=("arbitrary",),
            vmem_limit_bytes=128 * 1024 * 1024,
        ),
    )(jnp.stack([sidx, sidx]), sv_t, new_t)

    return (
        ct_out.T,
        ind_out,
        jnp.transpose(sv_out, (0, 2, 1)),
        save_index + 1,
    )
